# baseline (speedup 1.0000x reference)
import functools

import jax
import jax.numpy as jnp
from jax import lax
from jax.experimental import pallas as pl
from jax.experimental.pallas import tpu as pltpu
from jax.experimental.pallas import tpu_sc as plsc

_D = 64
_L = 128
_K = 2
_C = _K * _L
_NBUF = 4
_LA = 2


def _emb_body(nchunk, sbase, ids_hbm, table_hbm, out_hbm, idx_v, rows_v,
              gsem0, gsem1, gsem2, gsem3, wsem0, wsem1, wsem2, wsem3):
    wid = lax.axis_index("s") * 2 + lax.axis_index("c")
    bpw = nchunk * _C
    base = wid * bpw
    ids0 = sbase + base
    gsems = (gsem0, gsem1, gsem2, gsem3)
    wsems = (wsem0, wsem1, wsem2, wsem3)

    def out_slice(g):
        t0 = base + g * _C
        iblk = t0 // (2 * _BLK)
        u = t0 % (2 * _BLK)
        h = u // _BLK
        q = u % _BLK
        return pl.ds(iblk * _BLK + q, _C), pl.ds(h * _D, _D)

    def load_and_fire(b, g):
        pltpu.sync_copy(ids_hbm.at[pl.ds(ids0 + g * _C, _C)], idx_v.at[b])
        for j in range(_K):
            pltpu.async_copy(
                table_hbm.at[idx_v.at[b, pl.ds(j * _L, _L)]],
                rows_v.at[b, pl.ds(j * _L, _L), :],
                gsems[b])

    def drain_gather(b):
        pltpu.make_async_copy(
            table_hbm.at[pl.ds(0, _C), :], rows_v.at[b], gsems[b]).wait()

    def drain_write(b):
        pltpu.make_async_copy(
            table_hbm.at[pl.ds(0, _C), :], rows_v.at[b], wsems[b]).wait()

    for g in range(_LA):
        load_and_fire(g % _NBUF, g)

    @pl.loop(0, nchunk, step=_NBUF)
    def _chunks(g0):
        for b in range(_NBUF):
            g = g0 + b
            drain_gather(b)
            rs, cs = out_slice(g)
            pltpu.async_copy(rows_v.at[b], out_hbm.at[rs, cs], wsems[b])
            h = g + _LA
            b2 = (b + _LA) % _NBUF

            @pl.when(h < nchunk)
            def _refill():
                @pl.when(h >= _NBUF)
                def _free_buf():
                    drain_write(b2)

                load_and_fire(b2, h)

    for b in range(_NBUF):
        drain_write(b)


_BLK = 16384


def _transpose_body(x_ref, o_ref):
    x = x_ref[...]
    o_ref[:, :_BLK] = x[:, :_D].T
    o_ref[:, _BLK:] = x[:, _D:].T


def _transpose_upd_body(x_ref, z_ref, o_ref):
    del z_ref
    _transpose_body(x_ref, o_ref)


def _tc_transpose_slice(y2s, z_prev, e, s, ns):
    nb = (e // ns) // (2 * _BLK)
    out_spec = pl.BlockSpec((_D, 2 * _BLK), lambda i: (0, s * nb + i))
    in_spec = pl.BlockSpec((_BLK, 2 * _D), lambda i: (i, 0))
    out_shape = jax.ShapeDtypeStruct((_D, e), jnp.float32)
    params = pltpu.CompilerParams(vmem_limit_bytes=100 * 2**20)
    if z_prev is None:
        return pl.pallas_call(
            _transpose_body,
            grid=(nb,),
            in_specs=[in_spec],
            out_specs=out_spec,
            out_shape=out_shape,
            compiler_params=params,
        )(y2s)
    return pl.pallas_call(
        _transpose_upd_body,
        grid=(nb,),
        in_specs=[in_spec, pl.BlockSpec(memory_space=pl.ANY)],
        out_specs=out_spec,
        out_shape=out_shape,
        input_output_aliases={1: 0},
        compiler_params=params,
    )(y2s, z_prev)


_NSLICE = 10


def kernel(rel_ids, emb_weight):
    e = rel_ids.size
    nw = 32
    es = e // _NSLICE
    bpw = es // nw
    nchunk = bpw // _C
    assert bpw % (_C * _NBUF) == 0 and es % (2 * _BLK) == 0

    ids1d = rel_ids.reshape(-1).astype(jnp.int32)
    mesh = plsc.VectorSubcoreMesh(core_axis_name="c", subcore_axis_name="s")

    def sc_gather(s):
        run = pl.kernel(
            functools.partial(_emb_body, nchunk, s * es),
            out_type=jax.ShapeDtypeStruct((es // 2, 2 * _D), jnp.float32),
            mesh=mesh,
            scratch_types=[
                pltpu.VMEM((_NBUF, _C), jnp.int32),
                pltpu.VMEM((_NBUF, _C, _D), jnp.float32),
                pltpu.SemaphoreType.DMA,
                pltpu.SemaphoreType.DMA,
                pltpu.SemaphoreType.DMA,
                pltpu.SemaphoreType.DMA,
                pltpu.SemaphoreType.DMA,
                pltpu.SemaphoreType.DMA,
                pltpu.SemaphoreType.DMA,
                pltpu.SemaphoreType.DMA,
            ],
            compiler_params=pltpu.CompilerParams(use_tc_tiling_on_sc=False),
        )
        return run(ids1d, emb_weight)

    z = None
    for s in range(_NSLICE):
        y2s = sc_gather(s)
        z = _tc_transpose_slice(y2s, z, e, s, _NSLICE)
    return z.T


# speedup vs baseline: 1.0141x; 1.0141x over previous
import functools

import jax
import jax.numpy as jnp
from jax import lax
from jax.experimental import pallas as pl
from jax.experimental.pallas import tpu as pltpu
from jax.experimental.pallas import tpu_sc as plsc

_D = 64
_L = 128
_K = 2
_C = _K * _L
_NBUF = 4
_LA = 2


def _emb_body(nchunk, sbase, ids_hbm, table_hbm, out_hbm, idx_v, rows_v,
              gsem0, gsem1, gsem2, gsem3, wsem0, wsem1, wsem2, wsem3):
    wid = lax.axis_index("s") * 2 + lax.axis_index("c")
    bpw = nchunk * _C
    base = wid * bpw
    ids0 = sbase + base
    gsems = (gsem0, gsem1, gsem2, gsem3)
    wsems = (wsem0, wsem1, wsem2, wsem3)

    def out_slice(g):
        t0 = base + g * _C
        iblk = t0 // (2 * _BLK)
        u = t0 % (2 * _BLK)
        h = u // _BLK
        q = u % _BLK
        return pl.ds(iblk * _BLK + q, _C), pl.ds(h * _D, _D)

    def load_and_fire(b, g):
        pltpu.sync_copy(ids_hbm.at[pl.ds(ids0 + g * _C, _C)], idx_v.at[b])
        for j in range(_K):
            pltpu.async_copy(
                table_hbm.at[idx_v.at[b, pl.ds(j * _L, _L)]],
                rows_v.at[b, pl.ds(j * _L, _L), :],
                gsems[b])

    def drain_gather(b):
        pltpu.make_async_copy(
            table_hbm.at[pl.ds(0, _C), :], rows_v.at[b], gsems[b]).wait()

    def drain_write(b):
        pltpu.make_async_copy(
            table_hbm.at[pl.ds(0, _C), :], rows_v.at[b], wsems[b]).wait()

    for g in range(_LA):
        load_and_fire(g % _NBUF, g)

    @pl.loop(0, nchunk, step=_NBUF)
    def _chunks(g0):
        for b in range(_NBUF):
            g = g0 + b
            drain_gather(b)
            rs, cs = out_slice(g)
            pltpu.async_copy(rows_v.at[b], out_hbm.at[rs, cs], wsems[b])
            h = g + _LA
            b2 = (b + _LA) % _NBUF

            @pl.when(h < nchunk)
            def _refill():
                @pl.when(h >= _NBUF)
                def _free_buf():
                    drain_write(b2)

                load_and_fire(b2, h)

    for b in range(_NBUF):
        drain_write(b)


_BLK = 16384


def _transpose_body(x_ref, o_ref):
    x = x_ref[...]
    o_ref[:, :_BLK] = x[:, :_D].T
    o_ref[:, _BLK:] = x[:, _D:].T


def _transpose_upd_body(x_ref, z_ref, o_ref):
    del z_ref
    _transpose_body(x_ref, o_ref)


def _tc_transpose_slice(y2s, z_prev, e, s, ns):
    nb = (e // ns) // (2 * _BLK)
    out_spec = pl.BlockSpec((_D, 2 * _BLK), lambda i: (0, s * nb + i))
    in_spec = pl.BlockSpec((_BLK, 2 * _D), lambda i: (i, 0))
    out_shape = jax.ShapeDtypeStruct((_D, e), jnp.float32)
    params = pltpu.CompilerParams(vmem_limit_bytes=100 * 2**20)
    if z_prev is None:
        return pl.pallas_call(
            _transpose_body,
            grid=(nb,),
            in_specs=[in_spec],
            out_specs=out_spec,
            out_shape=out_shape,
            compiler_params=params,
        )(y2s)
    return pl.pallas_call(
        _transpose_upd_body,
        grid=(nb,),
        in_specs=[in_spec, pl.BlockSpec(memory_space=pl.ANY)],
        out_specs=out_spec,
        out_shape=out_shape,
        input_output_aliases={1: 0},
        compiler_params=params,
    )(y2s, z_prev)


_NSLICE = 5


def kernel(rel_ids, emb_weight):
    e = rel_ids.size
    nw = 32
    es = e // _NSLICE
    bpw = es // nw
    nchunk = bpw // _C
    assert bpw % (_C * _NBUF) == 0 and es % (2 * _BLK) == 0

    ids1d = rel_ids.reshape(-1).astype(jnp.int32)
    mesh = plsc.VectorSubcoreMesh(core_axis_name="c", subcore_axis_name="s")

    def sc_gather(s):
        run = pl.kernel(
            functools.partial(_emb_body, nchunk, s * es),
            out_type=jax.ShapeDtypeStruct((es // 2, 2 * _D), jnp.float32),
            mesh=mesh,
            scratch_types=[
                pltpu.VMEM((_NBUF, _C), jnp.int32),
                pltpu.VMEM((_NBUF, _C, _D), jnp.float32),
                pltpu.SemaphoreType.DMA,
                pltpu.SemaphoreType.DMA,
                pltpu.SemaphoreType.DMA,
                pltpu.SemaphoreType.DMA,
                pltpu.SemaphoreType.DMA,
                pltpu.SemaphoreType.DMA,
                pltpu.SemaphoreType.DMA,
                pltpu.SemaphoreType.DMA,
            ],
            compiler_params=pltpu.CompilerParams(use_tc_tiling_on_sc=False),
        )
        return run(ids1d, emb_weight)

    z = None
    for s in range(_NSLICE):
        y2s = sc_gather(s)
        z = _tc_transpose_slice(y2s, z, e, s, _NSLICE)
    return z.T
